# Initial kernel scaffold; baseline (speedup 1.0000x reference)
import jax
import jax.numpy as jnp
from jax.experimental import pallas as pl


def kernel(x, edge_index, edge_type, batch, W_rel1, W_root1, b1, gamma1, beta1, a1, W_rel2, W_root2, b2, gamma2, beta2, a2, fc_W, fc_b, out_W, out_b):
    raise NotImplementedError("write your pallas kernel here")


# speedup vs baseline: 2.4850x; 2.4850x over previous
import functools

import jax
import jax.numpy as jnp
from jax import lax
from jax.experimental import pallas as pl
from jax.experimental.pallas import tpu as pltpu

N = 50000
E = 800000
F_IN = 64
H = 64
R = 4
G = 64

NB = 1000
NBLK = N // NB
NPAD = 53248


def _seg_sums(x, src, dst, et):
    seg = et * NPAD + dst
    vals = x[src]
    S = jax.ops.segment_sum(vals, seg, num_segments=R * NPAD)
    cnt = jax.ops.segment_sum(jnp.ones((E,), jnp.float32), seg,
                              num_segments=R * NPAD)
    return S.reshape(R, NPAD, H), cnt.reshape(R, NPAD, 1)


def _layer_a_body(S_ref, cnt_ref, x_ref, Wrel_ref, Wroot_ref, b_ref,
                  hpre_ref, stats_ref):
    i = pl.program_id(0)
    acc = jnp.dot(x_ref[...], Wroot_ref[...],
                  preferred_element_type=jnp.float32) + b_ref[...]
    for r in range(R):
        inv = 1.0 / jnp.maximum(cnt_ref[r], 1.0)
        acc += jnp.dot(S_ref[r] * inv, Wrel_ref[r],
                       preferred_element_type=jnp.float32)
    hpre_ref[...] = acc

    @pl.when(i == 0)
    def _():
        stats_ref[...] = jnp.zeros_like(stats_ref)

    cs = jnp.sum(acc, axis=0).reshape(1, H)
    cq = jnp.sum(acc * acc, axis=0).reshape(1, H)
    stats_ref[...] += jnp.concatenate(
        [cs, cq, jnp.zeros((6, H), jnp.float32)], axis=0)


def _layer_a(S, cnt, x, W_rel, W_root, b):
    return pl.pallas_call(
        _layer_a_body,
        grid=(NBLK,),
        in_specs=[
            pl.BlockSpec((R, NB, H), lambda i: (0, i, 0)),
            pl.BlockSpec((R, NB, 1), lambda i: (0, i, 0)),
            pl.BlockSpec((NB, H), lambda i: (i, 0)),
            pl.BlockSpec((R, F_IN, H), lambda i: (0, 0, 0)),
            pl.BlockSpec((F_IN, H), lambda i: (0, 0)),
            pl.BlockSpec((1, H), lambda i: (0, 0)),
        ],
        out_specs=[
            pl.BlockSpec((NB, H), lambda i: (i, 0)),
            pl.BlockSpec((8, H), lambda i: (0, 0)),
        ],
        out_shape=[
            jax.ShapeDtypeStruct((N, H), jnp.float32),
            jax.ShapeDtypeStruct((8, H), jnp.float32),
        ],
    )(S, cnt, x, W_rel, W_root, b.reshape(1, H))


def _layer_c_body(hpre_ref, stats_ref, gamma_ref, beta_ref, a_ref, h_ref):
    s = stats_ref[...]
    mu = s[0:1, :] * (1.0 / N)
    ex2 = s[1:2, :] * (1.0 / N)
    var = ex2 - mu * mu
    y = (hpre_ref[...] - mu) * lax.rsqrt(var + 1e-5) * gamma_ref[...] \
        + beta_ref[...]
    a = a_ref[0, 0]
    y = jnp.where(y >= 0, y, a * y)
    nrm = jnp.sqrt(jnp.sum(y * y, axis=1, keepdims=True))
    h_ref[...] = y / jnp.maximum(nrm, 1e-12)


def _layer_c(hpre, stats, gamma, beta, a):
    return pl.pallas_call(
        _layer_c_body,
        grid=(NBLK,),
        in_specs=[
            pl.BlockSpec((NB, H), lambda i: (i, 0)),
            pl.BlockSpec((8, H), lambda i: (0, 0)),
            pl.BlockSpec((1, H), lambda i: (0, 0)),
            pl.BlockSpec((1, H), lambda i: (0, 0)),
            pl.BlockSpec((1, 1), lambda i: (0, 0)),
        ],
        out_specs=pl.BlockSpec((NB, H), lambda i: (i, 0)),
        out_shape=jax.ShapeDtypeStruct((N, H), jnp.float32),
    )(hpre, stats, gamma.reshape(1, H), beta.reshape(1, H), a.reshape(1, 1))


def _pool_body(batch_ref, h_ref, fcW_ref, fcb_ref, outW_ref, outb_ref,
               o_ref, P_acc, c_acc):
    i = pl.program_id(0)

    @pl.when(i == 0)
    def _():
        P_acc[...] = jnp.zeros_like(P_acc)
        c_acc[...] = jnp.zeros_like(c_acc)

    iota_g = lax.broadcasted_iota(jnp.int32, (1, G), 1)
    onehot = (batch_ref[...] == iota_g).astype(jnp.float32)
    P_acc[...] += lax.dot_general(onehot, h_ref[...],
                                  (((0,), (0,)), ((), ())),
                                  preferred_element_type=jnp.float32)
    c_acc[...] += lax.dot_general(onehot, jnp.ones((NB, 1), jnp.float32),
                                  (((0,), (0,)), ((), ())),
                                  preferred_element_type=jnp.float32)

    @pl.when(i == NBLK - 1)
    def _():
        pooled = P_acc[...] / jnp.maximum(c_acc[...], 1.0)
        z = jnp.dot(pooled, fcW_ref[...],
                    preferred_element_type=jnp.float32) + fcb_ref[...]
        z = jnp.maximum(z, 0.0)
        o_ref[...] = jnp.dot(z, outW_ref[...],
                             preferred_element_type=jnp.float32) + outb_ref[...]


def _pool(batch2d, h, fc_W, fc_b, out_W, out_b):
    return pl.pallas_call(
        _pool_body,
        grid=(NBLK,),
        in_specs=[
            pl.BlockSpec((NB, 1), lambda i: (i, 0)),
            pl.BlockSpec((NB, H), lambda i: (i, 0)),
            pl.BlockSpec((H, H), lambda i: (0, 0)),
            pl.BlockSpec((1, H), lambda i: (0, 0)),
            pl.BlockSpec((H, 1), lambda i: (0, 0)),
            pl.BlockSpec((1, 1), lambda i: (0, 0)),
        ],
        out_specs=pl.BlockSpec((G, 1), lambda i: (0, 0)),
        out_shape=jax.ShapeDtypeStruct((G, 1), jnp.float32),
        scratch_shapes=[
            pltpu.VMEM((G, H), jnp.float32),
            pltpu.VMEM((G, 1), jnp.float32),
        ],
    )(batch2d, h, fc_W, fc_b.reshape(1, H), out_W, out_b.reshape(1, 1))


def kernel(x, edge_index, edge_type, batch, W_rel1, W_root1, b1, gamma1,
           beta1, a1, W_rel2, W_root2, b2, gamma2, beta2, a2, fc_W, fc_b,
           out_W, out_b):
    src = edge_index[0]
    dst = edge_index[1]
    et = edge_type

    S1, cnt = _seg_sums(x, src, dst, et)
    hpre1, stats1 = _layer_a(S1, cnt, x, W_rel1, W_root1, b1)
    h1 = _layer_c(hpre1, stats1, gamma1, beta1, a1)

    S2, _ = _seg_sums(h1, src, dst, et)
    hpre2, stats2 = _layer_a(S2, cnt, h1, W_rel2, W_root2, b2)
    h2 = _layer_c(hpre2, stats2, gamma2, beta2, a2)

    return _pool(batch.reshape(N, 1), h2, fc_W, fc_b, out_W, out_b)


# speedup vs baseline: 4.1991x; 1.6897x over previous
import functools

import jax
import jax.numpy as jnp
from jax import lax
from jax.experimental import pallas as pl
from jax.experimental.pallas import tpu as pltpu
from jax.experimental.pallas import tpu_sc as plsc

N = 50000
E = 800000
F_IN = 64
H = 64
R = 4
G = 64

NB = 1000
NBLK = N // NB
NPAD = 53248

NC = 2
NS = 16
NW = NC * NS
L = 16

BN = 2048
BSH = 11
NBK = NPAD // BN
EC = E // NW
EHALF = 12512
ECP = 2 * EHALF
NVREG = EHALF // L
K = 256
NCH = 6
CAP = NCH * K
ACC_ROWS = R * BN + 128
SPARE = R * BN
ZR = ACC_ROWS // NS

_MESH = plsc.VectorSubcoreMesh(core_axis_name="c", subcore_axis_name="s",
                               num_cores=NC, num_subcores=NS)


def _sc1_body(srcT, dstT, etT, comp_src, comp_seg, counts,
              eb_src, eb_dst, eb_et, buf_src, buf_seg, cnt_v, sem):
    c = lax.axis_index("c")
    s = lax.axis_index("s")
    w = s * NC + c

    zero_v = jnp.zeros((L,), jnp.int32)
    spare_v = jnp.full((L,), SPARE, jnp.int32) + s

    def prefill(j, _):
        for b in range(NBK):
            buf_src[pl.ds(b * CAP + j * L, L)] = zero_v
            buf_seg[pl.ds(b * CAP + j * L, L)] = spare_v
        return 0

    lax.fori_loop(0, CAP // L, prefill, 0)

    ptrs = (jnp.zeros((), jnp.int32),) * NBK
    for half in range(2):
        e0 = w * ECP + half * EHALF
        pltpu.sync_copy(srcT.at[pl.ds(e0, EHALF)], eb_src)
        pltpu.sync_copy(dstT.at[pl.ds(e0, EHALF)], eb_dst)
        pltpu.sync_copy(etT.at[pl.ds(e0, EHALF)], eb_et)

        def scan(i, ptrs):
            dv = eb_dst[pl.ds(i * L, L)]
            sv = eb_src[pl.ds(i * L, L)]
            ev = eb_et[pl.ds(i * L, L)]
            bkt = lax.shift_right_logical(dv, BSH)
            lseg = jnp.bitwise_or(lax.shift_left(ev, BSH),
                                  jnp.bitwise_and(dv, BN - 1))
            out = []
            for b in range(NBK):
                m = bkt == b
                mi = m.astype(jnp.int32)
                inc = plsc.cumsum(mi)
                offs = (b * CAP + ptrs[b]) + (inc - mi)
                plsc.store_scatter(buf_src, [offs], sv, mask=m)
                plsc.store_scatter(buf_seg, [offs], lseg, mask=m)
                out.append(ptrs[b] + jnp.max(inc))
            return tuple(out)

        ptrs = lax.fori_loop(0, NVREG, scan, ptrs)

    lanes = lax.iota(jnp.int32, L)
    cv0 = jnp.zeros((L,), jnp.int32)
    cv1 = jnp.zeros((L,), jnp.int32)
    for b in range(NBK):
        pb = jnp.full((L,), 0, jnp.int32) + ptrs[b]
        if b < L:
            cv0 = jnp.where(lanes == b, pb, cv0)
        else:
            cv1 = jnp.where(lanes == (b - L), pb, cv1)
    cnt_v[pl.ds(0, L)] = cv0
    cnt_v[pl.ds(L, L)] = cv1
    pltpu.sync_copy(cnt_v, counts.at[pl.ds(w * 2 * L, 2 * L)])

    descs = []
    for b in range(NBK):
        o = (b * NW + w) * CAP
        descs.append(pltpu.async_copy(
            buf_src.at[pl.ds(b * CAP, CAP)], comp_src.at[pl.ds(o, CAP)], sem))
        descs.append(pltpu.async_copy(
            buf_seg.at[pl.ds(b * CAP, CAP)], comp_seg.at[pl.ds(o, CAP)], sem))
    for d in descs:
        d.wait()


def _compact_edges(srcT, dstT, etT):
    return pl.kernel(
        _sc1_body,
        out_type=[jax.ShapeDtypeStruct((NBK * NW * CAP,), jnp.int32),
                  jax.ShapeDtypeStruct((NBK * NW * CAP,), jnp.int32),
                  jax.ShapeDtypeStruct((NW * 2 * L,), jnp.int32)],
        mesh=_MESH,
        scratch_types=[
            pltpu.VMEM((EHALF,), jnp.int32),
            pltpu.VMEM((EHALF,), jnp.int32),
            pltpu.VMEM((EHALF,), jnp.int32),
            pltpu.VMEM((NBK * CAP,), jnp.int32),
            pltpu.VMEM((NBK * CAP,), jnp.int32),
            pltpu.VMEM((2 * L,), jnp.int32),
            pltpu.SemaphoreType.DMA,
        ],
        compiler_params=pltpu.CompilerParams(use_tc_tiling_on_sc=False, needs_layout_passes=False),
    )(srcT, dstT, etT)


def _sc2_body(comp_src, comp_seg, counts, vals, z_hbm, z1_hbm, outS, outC,
              idx_v, seg_v, rows_v, ones_v, zrow_v, zrow1_v, ct_v,
              acc, accC, sem):
    c = lax.axis_index("c")
    s = lax.axis_index("s")

    one_v = jnp.ones((L,), jnp.float32)
    for j in range(K // L):
        ones_v[pl.ds(j * L, L)] = one_v

    pltpu.sync_copy(z_hbm, zrow_v)
    pltpu.sync_copy(z1_hbm, zrow1_v)
    pltpu.sync_copy(counts.at[pl.ds(s * 2 * L, 2 * L)], ct_v.at[pl.ds(0, 2 * L)])
    pltpu.sync_copy(counts.at[pl.ds((s + NS) * 2 * L, 2 * L)],
                    ct_v.at[pl.ds(2 * L, 2 * L)])

    lanes = lax.iota(jnp.int32, L)

    for i in range((NBK + 1) // 2):
        b = 2 * i + c
        g = 1 if i >= 8 else 0
        lane = b - 16 * g

        @pl.when(b < NBK)
        def _():
            pltpu.sync_copy(zrow_v, acc.at[pl.ds(s * ZR, ZR)])
            pltpu.sync_copy(zrow1_v, accC.at[pl.ds(s * ZR, ZR)])
            plsc.subcore_barrier()

            for t_i in range(2):
                t = t_i * NS + s
                ctv = ct_v[pl.ds((2 * t_i + g) * L, L)]
                cnt = jnp.max(jnp.where(lanes == lane, ctv, 0))
                for j in range(NCH):
                    @pl.when(j * K < cnt)
                    def _():
                        o = ((b * NW + t) * NCH + j) * K
                        pltpu.sync_copy(comp_src.at[pl.ds(o, K)], idx_v)
                        pltpu.sync_copy(comp_seg.at[pl.ds(o, K)], seg_v)
                        pltpu.async_copy(vals.at[idx_v], rows_v, sem).wait()
                        pltpu.sync_copy(rows_v, acc.at[seg_v], add=True)
                        pltpu.sync_copy(ones_v, accC.at[seg_v], add=True)

            plsc.subcore_barrier()

            for r in range(R):
                a0 = r * BN + s * (BN // NS)
                o0 = r * NPAD + b * BN + s * (BN // NS)
                pltpu.sync_copy(acc.at[pl.ds(a0, BN // NS)],
                                outS.at[pl.ds(o0, BN // NS)])
                pltpu.sync_copy(accC.at[pl.ds(a0, BN // NS)],
                                outC.at[pl.ds(o0, BN // NS)])
            plsc.subcore_barrier()


def _seg_sums_sc(vals, comp_src, comp_seg, counts):
    z_hbm = jnp.zeros((ZR, H), jnp.float32)
    z1_hbm = jnp.zeros((ZR,), jnp.float32)
    S, cnt = pl.kernel(
        _sc2_body,
        out_type=[jax.ShapeDtypeStruct((R * NPAD, H), jnp.float32),
                  jax.ShapeDtypeStruct((R * NPAD,), jnp.float32)],
        mesh=_MESH,
        scratch_types=[
            pltpu.VMEM((K,), jnp.int32),
            pltpu.VMEM((K,), jnp.int32),
            pltpu.VMEM((K, H), jnp.float32),
            pltpu.VMEM((K,), jnp.float32),
            pltpu.VMEM((ZR, H), jnp.float32),
            pltpu.VMEM((ZR,), jnp.float32),
            pltpu.VMEM((4 * L,), jnp.int32),
            pltpu.VMEM_SHARED((ACC_ROWS, H), jnp.float32),
            pltpu.VMEM_SHARED((ACC_ROWS,), jnp.float32),
            pltpu.SemaphoreType.DMA,
        ],
        compiler_params=pltpu.CompilerParams(use_tc_tiling_on_sc=False, needs_layout_passes=False),
    )(comp_src, comp_seg, counts, vals, z_hbm, z1_hbm)
    return S.reshape(R, NPAD, H), cnt.reshape(R, NPAD, 1)


def _layer_a_body(S_ref, cnt_ref, x_ref, Wrel_ref, Wroot_ref, b_ref,
                  hpre_ref, stats_ref):
    i = pl.program_id(0)
    acc = jnp.dot(x_ref[...], Wroot_ref[...],
                  preferred_element_type=jnp.float32) + b_ref[...]
    for r in range(R):
        inv = 1.0 / jnp.maximum(cnt_ref[r], 1.0)
        acc += jnp.dot(S_ref[r] * inv, Wrel_ref[r],
                       preferred_element_type=jnp.float32)
    hpre_ref[...] = acc

    @pl.when(i == 0)
    def _():
        stats_ref[...] = jnp.zeros_like(stats_ref)

    cs = jnp.sum(acc, axis=0).reshape(1, H)
    cq = jnp.sum(acc * acc, axis=0).reshape(1, H)
    stats_ref[...] += jnp.concatenate(
        [cs, cq, jnp.zeros((6, H), jnp.float32)], axis=0)


def _layer_a(S, cnt, x, W_rel, W_root, b):
    return pl.pallas_call(
        _layer_a_body,
        grid=(NBLK,),
        in_specs=[
            pl.BlockSpec((R, NB, H), lambda i: (0, i, 0)),
            pl.BlockSpec((R, NB, 1), lambda i: (0, i, 0)),
            pl.BlockSpec((NB, H), lambda i: (i, 0)),
            pl.BlockSpec((R, F_IN, H), lambda i: (0, 0, 0)),
            pl.BlockSpec((F_IN, H), lambda i: (0, 0)),
            pl.BlockSpec((1, H), lambda i: (0, 0)),
        ],
        out_specs=[
            pl.BlockSpec((NB, H), lambda i: (i, 0)),
            pl.BlockSpec((8, H), lambda i: (0, 0)),
        ],
        out_shape=[
            jax.ShapeDtypeStruct((N, H), jnp.float32),
            jax.ShapeDtypeStruct((8, H), jnp.float32),
        ],
    )(S, cnt, x, W_rel, W_root, b.reshape(1, H))


def _layer_c_body(hpre_ref, stats_ref, gamma_ref, beta_ref, a_ref, h_ref):
    s = stats_ref[...]
    mu = s[0:1, :] * (1.0 / N)
    ex2 = s[1:2, :] * (1.0 / N)
    var = ex2 - mu * mu
    y = (hpre_ref[...] - mu) * lax.rsqrt(var + 1e-5) * gamma_ref[...] \
        + beta_ref[...]
    a = a_ref[0, 0]
    y = jnp.where(y >= 0, y, a * y)
    nrm = jnp.sqrt(jnp.sum(y * y, axis=1, keepdims=True))
    h_ref[...] = y / jnp.maximum(nrm, 1e-12)


def _layer_c(hpre, stats, gamma, beta, a):
    return pl.pallas_call(
        _layer_c_body,
        grid=(NBLK,),
        in_specs=[
            pl.BlockSpec((NB, H), lambda i: (i, 0)),
            pl.BlockSpec((8, H), lambda i: (0, 0)),
            pl.BlockSpec((1, H), lambda i: (0, 0)),
            pl.BlockSpec((1, H), lambda i: (0, 0)),
            pl.BlockSpec((1, 1), lambda i: (0, 0)),
        ],
        out_specs=pl.BlockSpec((NB, H), lambda i: (i, 0)),
        out_shape=jax.ShapeDtypeStruct((N, H), jnp.float32),
    )(hpre, stats, gamma.reshape(1, H), beta.reshape(1, H), a.reshape(1, 1))


def _pool_body(batch_ref, h_ref, fcW_ref, fcb_ref, outW_ref, outb_ref,
               o_ref, P_acc, c_acc):
    i = pl.program_id(0)

    @pl.when(i == 0)
    def _():
        P_acc[...] = jnp.zeros_like(P_acc)
        c_acc[...] = jnp.zeros_like(c_acc)

    iota_g = lax.broadcasted_iota(jnp.int32, (1, G), 1)
    onehot = (batch_ref[...] == iota_g).astype(jnp.float32)
    P_acc[...] += lax.dot_general(onehot, h_ref[...],
                                  (((0,), (0,)), ((), ())),
                                  preferred_element_type=jnp.float32)
    c_acc[...] += lax.dot_general(onehot, jnp.ones((NB, 1), jnp.float32),
                                  (((0,), (0,)), ((), ())),
                                  preferred_element_type=jnp.float32)

    @pl.when(i == NBLK - 1)
    def _():
        pooled = P_acc[...] / jnp.maximum(c_acc[...], 1.0)
        z = jnp.dot(pooled, fcW_ref[...],
                    preferred_element_type=jnp.float32) + fcb_ref[...]
        z = jnp.maximum(z, 0.0)
        o_ref[...] = jnp.dot(z, outW_ref[...],
                             preferred_element_type=jnp.float32) + outb_ref[...]


def _pool(batch2d, h, fc_W, fc_b, out_W, out_b):
    return pl.pallas_call(
        _pool_body,
        grid=(NBLK,),
        in_specs=[
            pl.BlockSpec((NB, 1), lambda i: (i, 0)),
            pl.BlockSpec((NB, H), lambda i: (i, 0)),
            pl.BlockSpec((H, H), lambda i: (0, 0)),
            pl.BlockSpec((1, H), lambda i: (0, 0)),
            pl.BlockSpec((H, 1), lambda i: (0, 0)),
            pl.BlockSpec((1, 1), lambda i: (0, 0)),
        ],
        out_specs=pl.BlockSpec((G, 1), lambda i: (0, 0)),
        out_shape=jax.ShapeDtypeStruct((G, 1), jnp.float32),
        scratch_shapes=[
            pltpu.VMEM((G, H), jnp.float32),
            pltpu.VMEM((G, 1), jnp.float32),
        ],
    )(batch2d, h, fc_W, fc_b.reshape(1, H), out_W, out_b.reshape(1, 1))


def _pad_chunks(a, fill):
    a2 = a.reshape(NW, EC)
    pad = jnp.full((NW, ECP - EC), fill, jnp.int32)
    return jnp.concatenate([a2, pad], axis=1).reshape(NW * ECP)


def kernel(x, edge_index, edge_type, batch, W_rel1, W_root1, b1, gamma1,
           beta1, a1, W_rel2, W_root2, b2, gamma2, beta2, a2, fc_W, fc_b,
           out_W, out_b):
    srcT = _pad_chunks(edge_index[0], 0)
    dstT = _pad_chunks(edge_index[1], 1 << 20)
    etT = _pad_chunks(edge_type, 0)

    comp_src, comp_seg, counts = _compact_edges(srcT, dstT, etT)

    S1, cnt = _seg_sums_sc(x, comp_src, comp_seg, counts)
    hpre1, stats1 = _layer_a(S1, cnt, x, W_rel1, W_root1, b1)
    h1 = _layer_c(hpre1, stats1, gamma1, beta1, a1)

    S2, _ = _seg_sums_sc(h1, comp_src, comp_seg, counts)
    hpre2, stats2 = _layer_a(S2, cnt, h1, W_rel2, W_root2, b2)
    h2 = _layer_c(hpre2, stats2, gamma2, beta2, a2)

    return _pool(batch.reshape(N, 1), h2, fc_W, fc_b, out_W, out_b)


# speedup vs baseline: 4.3419x; 1.0340x over previous
import functools

import jax
import jax.numpy as jnp
from jax import lax
from jax.experimental import pallas as pl
from jax.experimental.pallas import tpu as pltpu
from jax.experimental.pallas import tpu_sc as plsc

N = 50000
E = 800000
F_IN = 64
H = 64
R = 4
G = 64

NB = 1000
NBLK = N // NB
NPAD = 53248

NC = 2
NS = 16
NW = NC * NS
L = 16

BN = 2048
BSH = 11
NBK = NPAD // BN
EC = E // NW
EHALF = 12512
ECP = 2 * EHALF
NVREG = EHALF // L
K = 256
NCH = 6
CAP = NCH * K
ACC_ROWS = R * BN + 128
SPARE = R * BN
ZR = ACC_ROWS // NS

_MESH = plsc.VectorSubcoreMesh(core_axis_name="c", subcore_axis_name="s",
                               num_cores=NC, num_subcores=NS)


def _sc1_body(srcT, dstT, etT, comp_src, comp_seg, counts,
              eb_src, eb_dst, eb_et, buf_src, buf_seg, cnt_v, sem):
    c = lax.axis_index("c")
    s = lax.axis_index("s")
    w = s * NC + c

    zero_v = jnp.zeros((L,), jnp.int32)
    spare_v = jnp.full((L,), SPARE, jnp.int32) + s

    def prefill(j, _):
        for b in range(NBK):
            buf_src[pl.ds(b * CAP + j * L, L)] = zero_v
            buf_seg[pl.ds(b * CAP + j * L, L)] = spare_v
        return 0

    lax.fori_loop(0, CAP // L, prefill, 0)

    ptrs = (jnp.zeros((), jnp.int32),) * NBK
    for half in range(2):
        e0 = w * ECP + half * EHALF
        pltpu.sync_copy(srcT.at[pl.ds(e0, EHALF)], eb_src)
        pltpu.sync_copy(dstT.at[pl.ds(e0, EHALF)], eb_dst)
        pltpu.sync_copy(etT.at[pl.ds(e0, EHALF)], eb_et)

        def scan(i, ptrs):
            dv = eb_dst[pl.ds(i * L, L)]
            sv = eb_src[pl.ds(i * L, L)]
            ev = eb_et[pl.ds(i * L, L)]
            bkt = lax.shift_right_logical(dv, BSH)
            lseg = jnp.bitwise_or(lax.shift_left(ev, BSH),
                                  jnp.bitwise_and(dv, BN - 1))
            out = []
            for b in range(NBK):
                m = bkt == b
                mi = m.astype(jnp.int32)
                inc = plsc.cumsum(mi)
                offs = (b * CAP + ptrs[b]) + (inc - mi)
                plsc.store_scatter(buf_src, [offs], sv, mask=m)
                plsc.store_scatter(buf_seg, [offs], lseg, mask=m)
                out.append(ptrs[b] + jnp.max(inc))
            return tuple(out)

        ptrs = lax.fori_loop(0, NVREG, scan, ptrs)

    lanes = lax.iota(jnp.int32, L)
    cv0 = jnp.zeros((L,), jnp.int32)
    cv1 = jnp.zeros((L,), jnp.int32)
    for b in range(NBK):
        pb = jnp.full((L,), 0, jnp.int32) + ptrs[b]
        if b < L:
            cv0 = jnp.where(lanes == b, pb, cv0)
        else:
            cv1 = jnp.where(lanes == (b - L), pb, cv1)
    cnt_v[pl.ds(0, L)] = cv0
    cnt_v[pl.ds(L, L)] = cv1
    pltpu.sync_copy(cnt_v, counts.at[pl.ds(w * 2 * L, 2 * L)])

    descs = []
    for b in range(NBK):
        o = (b * NW + w) * CAP
        descs.append(pltpu.async_copy(
            buf_src.at[pl.ds(b * CAP, CAP)], comp_src.at[pl.ds(o, CAP)], sem))
        descs.append(pltpu.async_copy(
            buf_seg.at[pl.ds(b * CAP, CAP)], comp_seg.at[pl.ds(o, CAP)], sem))
    for d in descs:
        d.wait()


def _compact_edges(srcT, dstT, etT):
    return pl.kernel(
        _sc1_body,
        out_type=[jax.ShapeDtypeStruct((NBK * NW * CAP,), jnp.int32),
                  jax.ShapeDtypeStruct((NBK * NW * CAP,), jnp.int32),
                  jax.ShapeDtypeStruct((NW * 2 * L,), jnp.int32)],
        mesh=_MESH,
        scratch_types=[
            pltpu.VMEM((EHALF,), jnp.int32),
            pltpu.VMEM((EHALF,), jnp.int32),
            pltpu.VMEM((EHALF,), jnp.int32),
            pltpu.VMEM((NBK * CAP,), jnp.int32),
            pltpu.VMEM((NBK * CAP,), jnp.int32),
            pltpu.VMEM((2 * L,), jnp.int32),
            pltpu.SemaphoreType.DMA,
        ],
        compiler_params=pltpu.CompilerParams(use_tc_tiling_on_sc=False, needs_layout_passes=False),
    )(srcT, dstT, etT)


def _make_sc2_body(emit_counts):
    def body(*refs):
        if emit_counts:
            (comp_src, comp_seg, counts, vals, z_hbm, z1_hbm, outS, outC,
             idxA, idxB, segA, segB, rowsA, rowsB, ones_v, zrow_v, zrow1_v,
             ct_v, acc, accC, semI, semGA, semGB) = refs
        else:
            (comp_src, comp_seg, counts, vals, z_hbm, outS,
             idxA, idxB, segA, segB, rowsA, rowsB, zrow_v,
             ct_v, acc, semI, semGA, semGB) = refs
        c = lax.axis_index("c")
        s = lax.axis_index("s")
        idx = (idxA, idxB)
        seg = (segA, segB)
        rows = (rowsA, rowsB)
        semG = (semGA, semGB)

        if emit_counts:
            one_v = jnp.ones((L,), jnp.float32)
            for j in range(K // L):
                ones_v[pl.ds(j * L, L)] = one_v
            pltpu.sync_copy(z1_hbm, zrow1_v)
        pltpu.sync_copy(z_hbm, zrow_v)
        pltpu.sync_copy(counts.at[pl.ds(s * 2 * L, 2 * L)],
                        ct_v.at[pl.ds(0, 2 * L)])
        pltpu.sync_copy(counts.at[pl.ds((s + NS) * 2 * L, 2 * L)],
                        ct_v.at[pl.ds(2 * L, 2 * L)])

        lanes = lax.iota(jnp.int32, L)

        for i in range((NBK + 1) // 2):
            b = 2 * i + c
            g = 1 if i >= 8 else 0
            lane = b - 16 * g

            @pl.when(b < NBK)
            def _():
                pltpu.sync_copy(zrow_v, acc.at[pl.ds(s * ZR, ZR)])
                if emit_counts:
                    pltpu.sync_copy(zrow1_v, accC.at[pl.ds(s * ZR, ZR)])
                plsc.subcore_barrier()

                ct0 = ct_v[pl.ds(g * L, L)]
                ct1 = ct_v[pl.ds((2 + g) * L, L)]
                c0 = jnp.max(jnp.where(lanes == lane, ct0, 0))
                c1 = jnp.max(jnp.where(lanes == lane, ct1, 0))
                n0 = (c0 + (K - 1)) // K
                nq = n0 + (c1 + (K - 1)) // K
                base0 = (b * NW + s) * CAP
                base1 = (b * NW + NS + s) * CAP

                def off(q):
                    return jnp.where(q < n0, base0 + q * K,
                                     base1 + (q - n0) * K)

                def start_idx(q, p):
                    o = off(q)
                    pltpu.async_copy(comp_src.at[pl.ds(o, K)], idx[p], semI)
                    pltpu.async_copy(comp_seg.at[pl.ds(o, K)], seg[p], semI)

                def wait_idx(p):
                    pltpu.make_async_copy(comp_src.at[pl.ds(0, K)],
                                          idx[p], semI).wait()
                    pltpu.make_async_copy(comp_seg.at[pl.ds(0, K)],
                                          seg[p], semI).wait()

                def start_gather(p):
                    pltpu.async_copy(vals.at[idx[p]], rows[p], semG[p])

                def wait_gather(p):
                    pltpu.make_async_copy(vals.at[idx[p]], rows[p],
                                          semG[p]).wait()

                @pl.when(nq > 0)
                def _():
                    start_idx(0, 0)
                    wait_idx(0)
                    start_gather(0)

                @pl.when(nq > 1)
                def _():
                    start_idx(1, 1)

                for j in range(2 * NCH):
                    p = j % 2

                    @pl.when(j + 1 < nq)
                    def _():
                        wait_idx(1 - p)
                        start_gather(1 - p)

                    @pl.when(j < nq)
                    def _():
                        wait_gather(p)
                        pltpu.sync_copy(rows[p], acc.at[seg[p]], add=True)
                        if emit_counts:
                            pltpu.sync_copy(ones_v, accC.at[seg[p]],
                                            add=True)

                    @pl.when(j + 2 < nq)
                    def _():
                        start_idx(j + 2, p)

                plsc.subcore_barrier()

                for r in range(R):
                    a0 = r * BN + s * (BN // NS)
                    o0 = r * NPAD + b * BN + s * (BN // NS)
                    pltpu.sync_copy(acc.at[pl.ds(a0, BN // NS)],
                                    outS.at[pl.ds(o0, BN // NS)])
                    if emit_counts:
                        pltpu.sync_copy(accC.at[pl.ds(a0, BN // NS)],
                                        outC.at[pl.ds(o0, BN // NS)])
                plsc.subcore_barrier()

    return body


def _seg_sums_sc(vals, comp_src, comp_seg, counts, emit_counts):
    z_hbm = jnp.zeros((ZR, H), jnp.float32)
    cparams = pltpu.CompilerParams(use_tc_tiling_on_sc=False,
                                   needs_layout_passes=False)
    buf2 = lambda shape, dt: [pltpu.VMEM(shape, dt), pltpu.VMEM(shape, dt)]
    if emit_counts:
        z1_hbm = jnp.zeros((ZR,), jnp.float32)
        S, cnt = pl.kernel(
            _make_sc2_body(True),
            out_type=[jax.ShapeDtypeStruct((R * NPAD, H), jnp.float32),
                      jax.ShapeDtypeStruct((R * NPAD,), jnp.float32)],
            mesh=_MESH,
            scratch_types=(
                buf2((K,), jnp.int32) + buf2((K,), jnp.int32)
                + buf2((K, H), jnp.float32)
                + [pltpu.VMEM((K,), jnp.float32),
                   pltpu.VMEM((ZR, H), jnp.float32),
                   pltpu.VMEM((ZR,), jnp.float32),
                   pltpu.VMEM((4 * L,), jnp.int32),
                   pltpu.VMEM_SHARED((ACC_ROWS, H), jnp.float32),
                   pltpu.VMEM_SHARED((ACC_ROWS,), jnp.float32),
                   pltpu.SemaphoreType.DMA,
                   pltpu.SemaphoreType.DMA,
                   pltpu.SemaphoreType.DMA]),
            compiler_params=cparams,
        )(comp_src, comp_seg, counts, vals, z_hbm, z1_hbm)
        return S.reshape(R, NPAD, H), cnt.reshape(R, NPAD, 1)
    S = pl.kernel(
        _make_sc2_body(False),
        out_type=jax.ShapeDtypeStruct((R * NPAD, H), jnp.float32),
        mesh=_MESH,
        scratch_types=(
            buf2((K,), jnp.int32) + buf2((K,), jnp.int32)
            + buf2((K, H), jnp.float32)
            + [pltpu.VMEM((ZR, H), jnp.float32),
               pltpu.VMEM((4 * L,), jnp.int32),
               pltpu.VMEM_SHARED((ACC_ROWS, H), jnp.float32),
               pltpu.SemaphoreType.DMA,
               pltpu.SemaphoreType.DMA,
               pltpu.SemaphoreType.DMA]),
        compiler_params=cparams,
    )(comp_src, comp_seg, counts, vals, z_hbm)
    return S.reshape(R, NPAD, H), None


def _layer_a_body(S_ref, cnt_ref, x_ref, Wrel_ref, Wroot_ref, b_ref,
                  hpre_ref, stats_ref):
    i = pl.program_id(0)
    acc = jnp.dot(x_ref[...], Wroot_ref[...],
                  preferred_element_type=jnp.float32) + b_ref[...]
    for r in range(R):
        inv = 1.0 / jnp.maximum(cnt_ref[r], 1.0)
        acc += jnp.dot(S_ref[r] * inv, Wrel_ref[r],
                       preferred_element_type=jnp.float32)
    hpre_ref[...] = acc

    @pl.when(i == 0)
    def _():
        stats_ref[...] = jnp.zeros_like(stats_ref)

    cs = jnp.sum(acc, axis=0).reshape(1, H)
    cq = jnp.sum(acc * acc, axis=0).reshape(1, H)
    stats_ref[...] += jnp.concatenate(
        [cs, cq, jnp.zeros((6, H), jnp.float32)], axis=0)


def _layer_a(S, cnt, x, W_rel, W_root, b):
    return pl.pallas_call(
        _layer_a_body,
        grid=(NBLK,),
        in_specs=[
            pl.BlockSpec((R, NB, H), lambda i: (0, i, 0)),
            pl.BlockSpec((R, NB, 1), lambda i: (0, i, 0)),
            pl.BlockSpec((NB, H), lambda i: (i, 0)),
            pl.BlockSpec((R, F_IN, H), lambda i: (0, 0, 0)),
            pl.BlockSpec((F_IN, H), lambda i: (0, 0)),
            pl.BlockSpec((1, H), lambda i: (0, 0)),
        ],
        out_specs=[
            pl.BlockSpec((NB, H), lambda i: (i, 0)),
            pl.BlockSpec((8, H), lambda i: (0, 0)),
        ],
        out_shape=[
            jax.ShapeDtypeStruct((N, H), jnp.float32),
            jax.ShapeDtypeStruct((8, H), jnp.float32),
        ],
    )(S, cnt, x, W_rel, W_root, b.reshape(1, H))


def _layer_c_body(hpre_ref, stats_ref, gamma_ref, beta_ref, a_ref, h_ref):
    s = stats_ref[...]
    mu = s[0:1, :] * (1.0 / N)
    ex2 = s[1:2, :] * (1.0 / N)
    var = ex2 - mu * mu
    y = (hpre_ref[...] - mu) * lax.rsqrt(var + 1e-5) * gamma_ref[...] \
        + beta_ref[...]
    a = a_ref[0, 0]
    y = jnp.where(y >= 0, y, a * y)
    nrm = jnp.sqrt(jnp.sum(y * y, axis=1, keepdims=True))
    h_ref[...] = y / jnp.maximum(nrm, 1e-12)


def _layer_c(hpre, stats, gamma, beta, a):
    return pl.pallas_call(
        _layer_c_body,
        grid=(NBLK,),
        in_specs=[
            pl.BlockSpec((NB, H), lambda i: (i, 0)),
            pl.BlockSpec((8, H), lambda i: (0, 0)),
            pl.BlockSpec((1, H), lambda i: (0, 0)),
            pl.BlockSpec((1, H), lambda i: (0, 0)),
            pl.BlockSpec((1, 1), lambda i: (0, 0)),
        ],
        out_specs=pl.BlockSpec((NB, H), lambda i: (i, 0)),
        out_shape=jax.ShapeDtypeStruct((N, H), jnp.float32),
    )(hpre, stats, gamma.reshape(1, H), beta.reshape(1, H), a.reshape(1, 1))


def _pool_body(batch_ref, h_ref, fcW_ref, fcb_ref, outW_ref, outb_ref,
               o_ref, P_acc, c_acc):
    i = pl.program_id(0)

    @pl.when(i == 0)
    def _():
        P_acc[...] = jnp.zeros_like(P_acc)
        c_acc[...] = jnp.zeros_like(c_acc)

    iota_g = lax.broadcasted_iota(jnp.int32, (1, G), 1)
    onehot = (batch_ref[...] == iota_g).astype(jnp.float32)
    P_acc[...] += lax.dot_general(onehot, h_ref[...],
                                  (((0,), (0,)), ((), ())),
                                  preferred_element_type=jnp.float32)
    c_acc[...] += lax.dot_general(onehot, jnp.ones((NB, 1), jnp.float32),
                                  (((0,), (0,)), ((), ())),
                                  preferred_element_type=jnp.float32)

    @pl.when(i == NBLK - 1)
    def _():
        pooled = P_acc[...] / jnp.maximum(c_acc[...], 1.0)
        z = jnp.dot(pooled, fcW_ref[...],
                    preferred_element_type=jnp.float32) + fcb_ref[...]
        z = jnp.maximum(z, 0.0)
        o_ref[...] = jnp.dot(z, outW_ref[...],
                             preferred_element_type=jnp.float32) + outb_ref[...]


def _pool(batch2d, h, fc_W, fc_b, out_W, out_b):
    return pl.pallas_call(
        _pool_body,
        grid=(NBLK,),
        in_specs=[
            pl.BlockSpec((NB, 1), lambda i: (i, 0)),
            pl.BlockSpec((NB, H), lambda i: (i, 0)),
            pl.BlockSpec((H, H), lambda i: (0, 0)),
            pl.BlockSpec((1, H), lambda i: (0, 0)),
            pl.BlockSpec((H, 1), lambda i: (0, 0)),
            pl.BlockSpec((1, 1), lambda i: (0, 0)),
        ],
        out_specs=pl.BlockSpec((G, 1), lambda i: (0, 0)),
        out_shape=jax.ShapeDtypeStruct((G, 1), jnp.float32),
        scratch_shapes=[
            pltpu.VMEM((G, H), jnp.float32),
            pltpu.VMEM((G, 1), jnp.float32),
        ],
    )(batch2d, h, fc_W, fc_b.reshape(1, H), out_W, out_b.reshape(1, 1))


def _pad_chunks(a, fill):
    a2 = a.reshape(NW, EC)
    pad = jnp.full((NW, ECP - EC), fill, jnp.int32)
    return jnp.concatenate([a2, pad], axis=1).reshape(NW * ECP)


def kernel(x, edge_index, edge_type, batch, W_rel1, W_root1, b1, gamma1,
           beta1, a1, W_rel2, W_root2, b2, gamma2, beta2, a2, fc_W, fc_b,
           out_W, out_b):
    srcT = _pad_chunks(edge_index[0], 0)
    dstT = _pad_chunks(edge_index[1], 1 << 20)
    etT = _pad_chunks(edge_type, 0)

    comp_src, comp_seg, counts = _compact_edges(srcT, dstT, etT)

    S1, cnt = _seg_sums_sc(x, comp_src, comp_seg, counts, True)
    hpre1, stats1 = _layer_a(S1, cnt, x, W_rel1, W_root1, b1)
    h1 = _layer_c(hpre1, stats1, gamma1, beta1, a1)

    S2, _ = _seg_sums_sc(h1, comp_src, comp_seg, counts, False)
    hpre2, stats2 = _layer_a(S2, cnt, h1, W_rel2, W_root2, b2)
    h2 = _layer_c(hpre2, stats2, gamma2, beta2, a2)

    return _pool(batch.reshape(N, 1), h2, fc_W, fc_b, out_W, out_b)


# speedup vs baseline: 8.0910x; 1.8635x over previous
import functools

import jax
import jax.numpy as jnp
from jax import lax
from jax.experimental import pallas as pl
from jax.experimental.pallas import tpu as pltpu
from jax.experimental.pallas import tpu_sc as plsc

N = 50000
E = 800000
F_IN = 64
H = 64
R = 4
G = 64

NB = 1000
NBLK = N // NB
NPAD = 53248

NC = 2
NS = 16
NW = NC * NS
L = 16

BN = 2048
BSH = 11
NBK = NPAD // BN
EC = E // NW
EHALF = 12512
ECP = 2 * EHALF
NVREG = EHALF // L
K = 256
NCH = 6
CAP = NCH * K
ACC_ROWS = R * BN + 128
SPARE = R * BN
ZR = ACC_ROWS // NS

FB = 256
NFB = 196
NFBP = NPAD // FB
FPC = BN // FB
CAP2 = 4608
NCH2 = CAP2 // K
SINK2 = R * FB
AR2 = SINK2 + 32
ARW2 = AR2 * H
NPASS = (NFB + NW - 1) // NW

_MESH = plsc.VectorSubcoreMesh(core_axis_name="c", subcore_axis_name="s",
                               num_cores=NC, num_subcores=NS)


def _sc1_body(srcT, dstT, etT, comp_src, comp_seg, counts,
              eb_src, eb_dst, eb_et, buf_src, buf_seg, cnt_v, sem):
    c = lax.axis_index("c")
    s = lax.axis_index("s")
    w = s * NC + c

    zero_v = jnp.zeros((L,), jnp.int32)
    spare_v = jnp.full((L,), SPARE, jnp.int32) + s

    def prefill(j, _):
        for b in range(NBK):
            buf_src[pl.ds(b * CAP + j * L, L)] = zero_v
            buf_seg[pl.ds(b * CAP + j * L, L)] = spare_v
        return 0

    lax.fori_loop(0, CAP // L, prefill, 0)

    ptrs = (jnp.zeros((), jnp.int32),) * NBK
    for half in range(2):
        e0 = w * ECP + half * EHALF
        pltpu.sync_copy(srcT.at[pl.ds(e0, EHALF)], eb_src)
        pltpu.sync_copy(dstT.at[pl.ds(e0, EHALF)], eb_dst)
        pltpu.sync_copy(etT.at[pl.ds(e0, EHALF)], eb_et)

        def scan(i, ptrs):
            dv = eb_dst[pl.ds(i * L, L)]
            sv = eb_src[pl.ds(i * L, L)]
            ev = eb_et[pl.ds(i * L, L)]
            bkt = lax.shift_right_logical(dv, BSH)
            lseg = jnp.bitwise_or(lax.shift_left(ev, BSH),
                                  jnp.bitwise_and(dv, BN - 1))
            out = []
            for b in range(NBK):
                m = bkt == b
                mi = m.astype(jnp.int32)
                inc = plsc.cumsum(mi)
                offs = (b * CAP + ptrs[b]) + (inc - mi)
                plsc.store_scatter(buf_src, [offs], sv, mask=m)
                plsc.store_scatter(buf_seg, [offs], lseg, mask=m)
                out.append(ptrs[b] + jnp.max(inc))
            return tuple(out)

        ptrs = lax.fori_loop(0, NVREG, scan, ptrs)

    lanes = lax.iota(jnp.int32, L)
    cv0 = jnp.zeros((L,), jnp.int32)
    cv1 = jnp.zeros((L,), jnp.int32)
    for b in range(NBK):
        pb = jnp.full((L,), 0, jnp.int32) + ptrs[b]
        if b < L:
            cv0 = jnp.where(lanes == b, pb, cv0)
        else:
            cv1 = jnp.where(lanes == (b - L), pb, cv1)
    cnt_v[pl.ds(0, L)] = cv0
    cnt_v[pl.ds(L, L)] = cv1
    pltpu.sync_copy(cnt_v, counts.at[pl.ds(w * 2 * L, 2 * L)])

    descs = []
    for b in range(NBK):
        o = (b * NW + w) * CAP
        descs.append(pltpu.async_copy(
            buf_src.at[pl.ds(b * CAP, CAP)], comp_src.at[pl.ds(o, CAP)], sem))
        descs.append(pltpu.async_copy(
            buf_seg.at[pl.ds(b * CAP, CAP)], comp_seg.at[pl.ds(o, CAP)], sem))
    for d in descs:
        d.wait()


def _compact_edges(srcT, dstT, etT):
    return pl.kernel(
        _sc1_body,
        out_type=[jax.ShapeDtypeStruct((NBK * NW * CAP,), jnp.int32),
                  jax.ShapeDtypeStruct((NBK * NW * CAP,), jnp.int32),
                  jax.ShapeDtypeStruct((NW * 2 * L,), jnp.int32)],
        mesh=_MESH,
        scratch_types=[
            pltpu.VMEM((EHALF,), jnp.int32),
            pltpu.VMEM((EHALF,), jnp.int32),
            pltpu.VMEM((EHALF,), jnp.int32),
            pltpu.VMEM((NBK * CAP,), jnp.int32),
            pltpu.VMEM((NBK * CAP,), jnp.int32),
            pltpu.VMEM((2 * L,), jnp.int32),
            pltpu.SemaphoreType.DMA,
        ],
        compiler_params=pltpu.CompilerParams(use_tc_tiling_on_sc=False, needs_layout_passes=False),
    )(srcT, dstT, etT)


PSZ = 2048
NPIECE = NW * CAP // PSZ


def _sc1b_body(comp_src, comp_seg, fine_src, fine_seg, counts2,
               pb_src, pb_seg, bf_src, bf_seg, cnt_v, sem):
    c = lax.axis_index("c")
    s = lax.axis_index("s")
    tt = s * NC + c

    @pl.when(tt < NBK)
    def _():
        zero_v = jnp.zeros((L,), jnp.int32)
        sink_v = jnp.full((L,), SINK2, jnp.int32)

        def prefill(j, _):
            bf_src[pl.ds(j * L, L)] = zero_v
            bf_seg[pl.ds(j * L, L)] = sink_v
            return 0

        lax.fori_loop(0, FPC * CAP2 // L, prefill, 0)

        def piece(pc, ptrs):
            o = tt * (NW * CAP) + pc * PSZ
            pltpu.sync_copy(comp_src.at[pl.ds(o, PSZ)], pb_src)
            pltpu.sync_copy(comp_seg.at[pl.ds(o, PSZ)], pb_seg)

            def scan(i, ptrs):
                sv = pb_src[pl.ds(i * L, L)]
                gv = pb_seg[pl.ds(i * L, L)]
                valid = gv < (R * BN)
                fl = jnp.bitwise_and(lax.shift_right_logical(gv, 8), FPC - 1)
                lf = jnp.bitwise_or(
                    lax.shift_left(lax.shift_right_logical(gv, BSH), 8),
                    jnp.bitwise_and(gv, FB - 1))
                out = []
                for f in range(FPC):
                    m = jnp.logical_and(fl == f, valid)
                    mi = m.astype(jnp.int32)
                    inc = plsc.cumsum(mi)
                    offs = (f * CAP2 + ptrs[f]) + (inc - mi)
                    plsc.store_scatter(bf_src, [offs], sv, mask=m)
                    plsc.store_scatter(bf_seg, [offs], lf, mask=m)
                    out.append(ptrs[f] + jnp.max(inc))
                return tuple(out)

            return lax.fori_loop(0, PSZ // L, scan, ptrs)

        ptrs = lax.fori_loop(0, NPIECE, piece,
                             (jnp.zeros((), jnp.int32),) * FPC)

        lanes = lax.iota(jnp.int32, L)
        cv = jnp.zeros((L,), jnp.int32)
        for f in range(FPC):
            cv = jnp.where(lanes == f, jnp.zeros((L,), jnp.int32) + ptrs[f],
                           cv)
        cnt_v[pl.ds(0, L)] = cv
        pltpu.sync_copy(cnt_v.at[pl.ds(0, 8)], counts2.at[pl.ds(tt * 8, 8)])

        descs = []
        for f in range(FPC):
            fb = tt * FPC + f
            descs.append(pltpu.async_copy(
                bf_src.at[pl.ds(f * CAP2, CAP2)],
                fine_src.at[pl.ds(fb * CAP2, CAP2)], sem))
            descs.append(pltpu.async_copy(
                bf_seg.at[pl.ds(f * CAP2, CAP2)],
                fine_seg.at[pl.ds(fb * CAP2, CAP2)], sem))
        for d in descs:
            d.wait()


def _partition_fine(comp_src, comp_seg):
    return pl.kernel(
        _sc1b_body,
        out_type=[jax.ShapeDtypeStruct((NFBP * CAP2,), jnp.int32),
                  jax.ShapeDtypeStruct((NFBP * CAP2,), jnp.int32),
                  jax.ShapeDtypeStruct((NBK * FPC,), jnp.int32)],
        mesh=_MESH,
        scratch_types=[
            pltpu.VMEM((PSZ,), jnp.int32),
            pltpu.VMEM((PSZ,), jnp.int32),
            pltpu.VMEM((FPC * CAP2,), jnp.int32),
            pltpu.VMEM((FPC * CAP2,), jnp.int32),
            pltpu.VMEM((L,), jnp.int32),
            pltpu.SemaphoreType.DMA,
        ],
        compiler_params=pltpu.CompilerParams(use_tc_tiling_on_sc=False,
                                             needs_layout_passes=False),
    )(comp_src, comp_seg)


def _make_sc2_body(emit_counts):
    def body(*refs):
        if emit_counts:
            (fine_src, fine_seg, counts2, vals, outS, outC,
             idxA, idxB, segA, segB, rowsA, rowsB, ctb, acc, accC,
             semIA, semIB, semGA, semGB) = refs
        else:
            (fine_src, fine_seg, counts2, vals, outS,
             idxA, idxB, segA, segB, rowsA, rowsB, ctb, acc,
             semIA, semIB, semGA, semGB) = refs
        c = lax.axis_index("c")
        s = lax.axis_index("s")
        w = s * NC + c
        idx = (idxA, idxB)
        seg = (segA, segB)
        rows = (rowsA, rowsB)
        semI = (semIA, semIB)
        semG = (semGA, semGB)

        pltpu.sync_copy(counts2, ctb.at[pl.ds(0, NBK * FPC)])
        lanes = lax.iota(jnp.int32, L)
        lane = jnp.bitwise_and(w, L - 1)
        half = s >= (NS // 2)
        lanemask = lanes == lane
        e1_v = jnp.where(lanes == 0, jnp.float32(1.0), jnp.float32(0.0))

        for ps in range(NPASS):
            fb = ps * NW + w

            @pl.when(fb < NFB)
            def _():
                v1 = ctb[pl.ds(ps * NW, L)]
                v2 = ctb[pl.ds(ps * NW + L, L)]
                cv = jnp.where(half, v2, v1)
                cnt = jnp.max(jnp.where(lanemask, cv, 0))
                nch = (cnt + (K - 1)) // K
                base = fb * CAP2

                zv = jnp.zeros((L,), jnp.float32)

                def zloop(i, _):
                    acc[pl.ds(i * L, L)] = zv
                    return 0

                lax.fori_loop(0, ARW2 // L, zloop, 0)
                if emit_counts:
                    def zloop2(i, _):
                        accC[pl.ds(i * L, L)] = zv
                        return 0

                    lax.fori_loop(0, (AR2 + L) // L, zloop2, 0)

                def start_is(q, p):
                    o = base + q * K
                    pltpu.async_copy(fine_src.at[pl.ds(o, K)], idx[p],
                                     semI[p])
                    pltpu.async_copy(fine_seg.at[pl.ds(o, K)], seg[p],
                                     semI[p])

                def wait_is(p):
                    pltpu.make_async_copy(fine_src.at[pl.ds(0, K)], idx[p],
                                          semI[p]).wait()
                    pltpu.make_async_copy(fine_seg.at[pl.ds(0, K)], seg[p],
                                          semI[p]).wait()

                def start_g(p):
                    pltpu.async_copy(vals.at[idx[p]], rows[p], semG[p])

                def wait_g(p):
                    pltpu.make_async_copy(vals.at[idx[p]], rows[p],
                                          semG[p]).wait()

                def add_chunk(p):
                    def group(g, _):
                        sg = seg[p][pl.ds(g * L, L)]
                        so = sg * H
                        for ln in range(L):
                            o = so[ln]
                            e = g * L + ln
                            for j in range(H // L):
                                v = rows[p][e, pl.ds(j * L, L)]
                                plsc.addupdate(
                                    acc.at[pl.ds(o + j * L, L)], v)
                            if emit_counts:
                                plsc.addupdate(
                                    accC.at[pl.ds(sg[ln], L)], e1_v)
                        return 0

                    lax.fori_loop(0, K // L, group, 0)

                @pl.when(nch > 0)
                def _():
                    start_is(0, 0)
                    wait_is(0)
                    start_g(0)

                @pl.when(nch > 1)
                def _():
                    start_is(1, 1)

                def piped(k2, _):
                    q = 2 * k2

                    @pl.when(q + 1 < nch)
                    def _():
                        wait_is(1)
                        start_g(1)

                    wait_g(0)
                    add_chunk(0)

                    @pl.when(q + 2 < nch)
                    def _():
                        start_is(q + 2, 0)
                        wait_is(0)
                        start_g(0)

                    @pl.when(q + 1 < nch)
                    def _():
                        wait_g(1)
                        add_chunk(1)

                    @pl.when(q + 3 < nch)
                    def _():
                        start_is(q + 3, 1)

                    return 0

                lax.fori_loop(0, (nch + 1) // 2, piped, 0)

                for r in range(R):
                    pltpu.sync_copy(
                        acc.at[pl.ds(r * FB * H, FB * H)],
                        outS.at[pl.ds((r * NPAD + fb * FB) * H, FB * H)])
                    if emit_counts:
                        pltpu.sync_copy(
                            accC.at[pl.ds(r * FB, FB)],
                            outC.at[pl.ds(r * NPAD + fb * FB, FB)])

    return body


def _seg_sums_sc(vals, fine_src, fine_seg, counts2, emit_counts):
    cparams = pltpu.CompilerParams(use_tc_tiling_on_sc=False,
                                   needs_layout_passes=False)
    buf2 = lambda shape, dt: [pltpu.VMEM(shape, dt), pltpu.VMEM(shape, dt)]
    sems = [pltpu.SemaphoreType.DMA] * 4
    if emit_counts:
        S, cnt = pl.kernel(
            _make_sc2_body(True),
            out_type=[jax.ShapeDtypeStruct((R * NPAD * H,), jnp.float32),
                      jax.ShapeDtypeStruct((R * NPAD,), jnp.float32)],
            mesh=_MESH,
            scratch_types=(
                buf2((K,), jnp.int32) + buf2((K,), jnp.int32)
                + buf2((K, H), jnp.float32)
                + [pltpu.VMEM((NPASS * NW,), jnp.int32),
                   pltpu.VMEM((ARW2,), jnp.float32),
                   pltpu.VMEM((AR2 + L,), jnp.float32)]
                + sems),
            compiler_params=cparams,
        )(fine_src, fine_seg, counts2, vals)
        return (S.reshape(R, NPAD, H), cnt.reshape(R, NPAD, 1))
    S = pl.kernel(
        _make_sc2_body(False),
        out_type=jax.ShapeDtypeStruct((R * NPAD * H,), jnp.float32),
        mesh=_MESH,
        scratch_types=(
            buf2((K,), jnp.int32) + buf2((K,), jnp.int32)
            + buf2((K, H), jnp.float32)
            + [pltpu.VMEM((NPASS * NW,), jnp.int32),
               pltpu.VMEM((ARW2,), jnp.float32)]
            + sems),
        compiler_params=cparams,
    )(fine_src, fine_seg, counts2, vals)
    return S.reshape(R, NPAD, H), None


def _layer_a_body(S_ref, cnt_ref, x_ref, Wrel_ref, Wroot_ref, b_ref,
                  hpre_ref, stats_ref):
    i = pl.program_id(0)
    acc = jnp.dot(x_ref[...], Wroot_ref[...],
                  preferred_element_type=jnp.float32) + b_ref[...]
    for r in range(R):
        inv = 1.0 / jnp.maximum(cnt_ref[r], 1.0)
        acc += jnp.dot(S_ref[r] * inv, Wrel_ref[r],
                       preferred_element_type=jnp.float32)
    hpre_ref[...] = acc

    @pl.when(i == 0)
    def _():
        stats_ref[...] = jnp.zeros_like(stats_ref)

    cs = jnp.sum(acc, axis=0).reshape(1, H)
    cq = jnp.sum(acc * acc, axis=0).reshape(1, H)
    stats_ref[...] += jnp.concatenate(
        [cs, cq, jnp.zeros((6, H), jnp.float32)], axis=0)


def _layer_a(S, cnt, x, W_rel, W_root, b):
    return pl.pallas_call(
        _layer_a_body,
        grid=(NBLK,),
        in_specs=[
            pl.BlockSpec((R, NB, H), lambda i: (0, i, 0)),
            pl.BlockSpec((R, NB, 1), lambda i: (0, i, 0)),
            pl.BlockSpec((NB, H), lambda i: (i, 0)),
            pl.BlockSpec((R, F_IN, H), lambda i: (0, 0, 0)),
            pl.BlockSpec((F_IN, H), lambda i: (0, 0)),
            pl.BlockSpec((1, H), lambda i: (0, 0)),
        ],
        out_specs=[
            pl.BlockSpec((NB, H), lambda i: (i, 0)),
            pl.BlockSpec((8, H), lambda i: (0, 0)),
        ],
        out_shape=[
            jax.ShapeDtypeStruct((N, H), jnp.float32),
            jax.ShapeDtypeStruct((8, H), jnp.float32),
        ],
    )(S, cnt, x, W_rel, W_root, b.reshape(1, H))


def _layer_c_body(hpre_ref, stats_ref, gamma_ref, beta_ref, a_ref, h_ref):
    s = stats_ref[...]
    mu = s[0:1, :] * (1.0 / N)
    ex2 = s[1:2, :] * (1.0 / N)
    var = ex2 - mu * mu
    y = (hpre_ref[...] - mu) * lax.rsqrt(var + 1e-5) * gamma_ref[...] \
        + beta_ref[...]
    a = a_ref[0, 0]
    y = jnp.where(y >= 0, y, a * y)
    nrm = jnp.sqrt(jnp.sum(y * y, axis=1, keepdims=True))
    h_ref[...] = y / jnp.maximum(nrm, 1e-12)


def _layer_c(hpre, stats, gamma, beta, a):
    return pl.pallas_call(
        _layer_c_body,
        grid=(NBLK,),
        in_specs=[
            pl.BlockSpec((NB, H), lambda i: (i, 0)),
            pl.BlockSpec((8, H), lambda i: (0, 0)),
            pl.BlockSpec((1, H), lambda i: (0, 0)),
            pl.BlockSpec((1, H), lambda i: (0, 0)),
            pl.BlockSpec((1, 1), lambda i: (0, 0)),
        ],
        out_specs=pl.BlockSpec((NB, H), lambda i: (i, 0)),
        out_shape=jax.ShapeDtypeStruct((N, H), jnp.float32),
    )(hpre, stats, gamma.reshape(1, H), beta.reshape(1, H), a.reshape(1, 1))


def _pool_body(batch_ref, h_ref, fcW_ref, fcb_ref, outW_ref, outb_ref,
               o_ref, P_acc, c_acc):
    i = pl.program_id(0)

    @pl.when(i == 0)
    def _():
        P_acc[...] = jnp.zeros_like(P_acc)
        c_acc[...] = jnp.zeros_like(c_acc)

    iota_g = lax.broadcasted_iota(jnp.int32, (1, G), 1)
    onehot = (batch_ref[...] == iota_g).astype(jnp.float32)
    P_acc[...] += lax.dot_general(onehot, h_ref[...],
                                  (((0,), (0,)), ((), ())),
                                  preferred_element_type=jnp.float32)
    c_acc[...] += lax.dot_general(onehot, jnp.ones((NB, 1), jnp.float32),
                                  (((0,), (0,)), ((), ())),
                                  preferred_element_type=jnp.float32)

    @pl.when(i == NBLK - 1)
    def _():
        pooled = P_acc[...] / jnp.maximum(c_acc[...], 1.0)
        z = jnp.dot(pooled, fcW_ref[...],
                    preferred_element_type=jnp.float32) + fcb_ref[...]
        z = jnp.maximum(z, 0.0)
        o_ref[...] = jnp.dot(z, outW_ref[...],
                             preferred_element_type=jnp.float32) + outb_ref[...]


def _pool(batch2d, h, fc_W, fc_b, out_W, out_b):
    return pl.pallas_call(
        _pool_body,
        grid=(NBLK,),
        in_specs=[
            pl.BlockSpec((NB, 1), lambda i: (i, 0)),
            pl.BlockSpec((NB, H), lambda i: (i, 0)),
            pl.BlockSpec((H, H), lambda i: (0, 0)),
            pl.BlockSpec((1, H), lambda i: (0, 0)),
            pl.BlockSpec((H, 1), lambda i: (0, 0)),
            pl.BlockSpec((1, 1), lambda i: (0, 0)),
        ],
        out_specs=pl.BlockSpec((G, 1), lambda i: (0, 0)),
        out_shape=jax.ShapeDtypeStruct((G, 1), jnp.float32),
        scratch_shapes=[
            pltpu.VMEM((G, H), jnp.float32),
            pltpu.VMEM((G, 1), jnp.float32),
        ],
    )(batch2d, h, fc_W, fc_b.reshape(1, H), out_W, out_b.reshape(1, 1))


def _pad_chunks(a, fill):
    a2 = a.reshape(NW, EC)
    pad = jnp.full((NW, ECP - EC), fill, jnp.int32)
    return jnp.concatenate([a2, pad], axis=1).reshape(NW * ECP)


def kernel(x, edge_index, edge_type, batch, W_rel1, W_root1, b1, gamma1,
           beta1, a1, W_rel2, W_root2, b2, gamma2, beta2, a2, fc_W, fc_b,
           out_W, out_b):
    srcT = _pad_chunks(edge_index[0], 0)
    dstT = _pad_chunks(edge_index[1], 1 << 20)
    etT = _pad_chunks(edge_type, 0)

    comp_src, comp_seg, _counts = _compact_edges(srcT, dstT, etT)
    fine_src, fine_seg, counts2 = _partition_fine(comp_src, comp_seg)

    S1, cnt = _seg_sums_sc(x, fine_src, fine_seg, counts2, True)
    hpre1, stats1 = _layer_a(S1, cnt, x, W_rel1, W_root1, b1)
    h1 = _layer_c(hpre1, stats1, gamma1, beta1, a1)

    S2, _ = _seg_sums_sc(h1, fine_src, fine_seg, counts2, False)
    hpre2, stats2 = _layer_a(S2, cnt, h1, W_rel2, W_root2, b2)
    h2 = _layer_c(hpre2, stats2, gamma2, beta2, a2)

    return _pool(batch.reshape(N, 1), h2, fc_W, fc_b, out_W, out_b)


# speedup vs baseline: 8.8141x; 1.0894x over previous
import functools

import jax
import jax.numpy as jnp
from jax import lax
from jax.experimental import pallas as pl
from jax.experimental.pallas import tpu as pltpu
from jax.experimental.pallas import tpu_sc as plsc

N = 50000
E = 800000
F_IN = 64
H = 64
R = 4
G = 64

NB = 1000
NBLK = N // NB
NPAD = 53248

NC = 2
NS = 16
NW = NC * NS
L = 16

BN = 2048
BSH = 11
NBK = NPAD // BN
EC = E // NW
EHALF = 12512
ECP = 2 * EHALF
NVREG = EHALF // L
K = 256
NCH = 6
CAP = NCH * K
ACC_ROWS = R * BN + 128
SPARE = R * BN
ZR = ACC_ROWS // NS

FB = 256
NFB = 196
NFBP = NPAD // FB
FPC = BN // FB
CAP2 = 4608
NCH2 = CAP2 // K
SINK2 = R * FB
AR2 = SINK2 + 32
ARW2 = AR2 * H
NPASS = (NFB + NW - 1) // NW

_MESH = plsc.VectorSubcoreMesh(core_axis_name="c", subcore_axis_name="s",
                               num_cores=NC, num_subcores=NS)


def _sc1_body(srcT, dstT, etT, comp_src, comp_seg, counts,
              eb_src, eb_dst, eb_et, buf_src, buf_seg, cnt_v, sem):
    c = lax.axis_index("c")
    s = lax.axis_index("s")
    w = s * NC + c

    zero_v = jnp.zeros((L,), jnp.int32)
    spare_v = jnp.full((L,), SPARE, jnp.int32) + s

    def prefill(j, _):
        for b in range(NBK):
            buf_src[pl.ds(b * CAP + j * L, L)] = zero_v
            buf_seg[pl.ds(b * CAP + j * L, L)] = spare_v
        return 0

    lax.fori_loop(0, CAP // L, prefill, 0)

    ptrs = (jnp.zeros((), jnp.int32),) * NBK
    for half in range(2):
        e0 = w * ECP + half * EHALF
        pltpu.sync_copy(srcT.at[pl.ds(e0, EHALF)], eb_src)
        pltpu.sync_copy(dstT.at[pl.ds(e0, EHALF)], eb_dst)
        pltpu.sync_copy(etT.at[pl.ds(e0, EHALF)], eb_et)

        def scan(i, ptrs):
            dv = eb_dst[pl.ds(i * L, L)]
            sv = eb_src[pl.ds(i * L, L)]
            ev = eb_et[pl.ds(i * L, L)]
            bkt = lax.shift_right_logical(dv, BSH)
            lseg = jnp.bitwise_or(lax.shift_left(ev, BSH),
                                  jnp.bitwise_and(dv, BN - 1))
            out = []
            for b in range(NBK):
                m = bkt == b
                mi = m.astype(jnp.int32)
                inc = plsc.cumsum(mi)
                offs = (b * CAP + ptrs[b]) + (inc - mi)
                plsc.store_scatter(buf_src, [offs], sv, mask=m)
                plsc.store_scatter(buf_seg, [offs], lseg, mask=m)
                out.append(ptrs[b] + jnp.max(inc))
            return tuple(out)

        ptrs = lax.fori_loop(0, NVREG, scan, ptrs)

    lanes = lax.iota(jnp.int32, L)
    cv0 = jnp.zeros((L,), jnp.int32)
    cv1 = jnp.zeros((L,), jnp.int32)
    for b in range(NBK):
        pb = jnp.full((L,), 0, jnp.int32) + ptrs[b]
        if b < L:
            cv0 = jnp.where(lanes == b, pb, cv0)
        else:
            cv1 = jnp.where(lanes == (b - L), pb, cv1)
    cnt_v[pl.ds(0, L)] = cv0
    cnt_v[pl.ds(L, L)] = cv1
    pltpu.sync_copy(cnt_v, counts.at[pl.ds(w * 2 * L, 2 * L)])

    descs = []
    for b in range(NBK):
        o = (b * NW + w) * CAP
        descs.append(pltpu.async_copy(
            buf_src.at[pl.ds(b * CAP, CAP)], comp_src.at[pl.ds(o, CAP)], sem))
        descs.append(pltpu.async_copy(
            buf_seg.at[pl.ds(b * CAP, CAP)], comp_seg.at[pl.ds(o, CAP)], sem))
    for d in descs:
        d.wait()


def _compact_edges(srcT, dstT, etT):
    return pl.kernel(
        _sc1_body,
        out_type=[jax.ShapeDtypeStruct((NBK * NW * CAP,), jnp.int32),
                  jax.ShapeDtypeStruct((NBK * NW * CAP,), jnp.int32),
                  jax.ShapeDtypeStruct((NW * 2 * L,), jnp.int32)],
        mesh=_MESH,
        scratch_types=[
            pltpu.VMEM((EHALF,), jnp.int32),
            pltpu.VMEM((EHALF,), jnp.int32),
            pltpu.VMEM((EHALF,), jnp.int32),
            pltpu.VMEM((NBK * CAP,), jnp.int32),
            pltpu.VMEM((NBK * CAP,), jnp.int32),
            pltpu.VMEM((2 * L,), jnp.int32),
            pltpu.SemaphoreType.DMA,
        ],
        compiler_params=pltpu.CompilerParams(use_tc_tiling_on_sc=False, needs_layout_passes=False),
    )(srcT, dstT, etT)


PSZ = 2048
NPIECE = NW * CAP // PSZ


def _sc1b_body(comp_src, comp_seg, fine_src, fine_seg, counts2,
               pb_src, pb_seg, bf_src, bf_seg, cnt_v, sem):
    c = lax.axis_index("c")
    s = lax.axis_index("s")
    tt = s * NC + c

    @pl.when(tt < NBK)
    def _():
        zero_v = jnp.zeros((L,), jnp.int32)
        sink_v = jnp.full((L,), SINK2, jnp.int32)

        def prefill(j, _):
            bf_src[pl.ds(j * L, L)] = zero_v
            bf_seg[pl.ds(j * L, L)] = sink_v
            return 0

        lax.fori_loop(0, FPC * CAP2 // L, prefill, 0)

        def piece(pc, ptrs):
            o = tt * (NW * CAP) + pc * PSZ
            pltpu.sync_copy(comp_src.at[pl.ds(o, PSZ)], pb_src)
            pltpu.sync_copy(comp_seg.at[pl.ds(o, PSZ)], pb_seg)

            def scan(i, ptrs):
                sv = pb_src[pl.ds(i * L, L)]
                gv = pb_seg[pl.ds(i * L, L)]
                valid = gv < (R * BN)
                fl = jnp.bitwise_and(lax.shift_right_logical(gv, 8), FPC - 1)
                lf = jnp.bitwise_or(
                    lax.shift_left(lax.shift_right_logical(gv, BSH), 8),
                    jnp.bitwise_and(gv, FB - 1))
                out = []
                for f in range(FPC):
                    m = jnp.logical_and(fl == f, valid)
                    mi = m.astype(jnp.int32)
                    inc = plsc.cumsum(mi)
                    offs = (f * CAP2 + ptrs[f]) + (inc - mi)
                    plsc.store_scatter(bf_src, [offs], sv, mask=m)
                    plsc.store_scatter(bf_seg, [offs], lf, mask=m)
                    out.append(ptrs[f] + jnp.max(inc))
                return tuple(out)

            return lax.fori_loop(0, PSZ // L, scan, ptrs)

        ptrs = lax.fori_loop(0, NPIECE, piece,
                             (jnp.zeros((), jnp.int32),) * FPC)

        lanes = lax.iota(jnp.int32, L)
        cv = jnp.zeros((L,), jnp.int32)
        for f in range(FPC):
            cv = jnp.where(lanes == f, jnp.zeros((L,), jnp.int32) + ptrs[f],
                           cv)
        cnt_v[pl.ds(0, L)] = cv
        pltpu.sync_copy(cnt_v.at[pl.ds(0, 8)], counts2.at[pl.ds(tt * 8, 8)])

        descs = []
        for f in range(FPC):
            fb = tt * FPC + f
            descs.append(pltpu.async_copy(
                bf_src.at[pl.ds(f * CAP2, CAP2)],
                fine_src.at[pl.ds(fb * CAP2, CAP2)], sem))
            descs.append(pltpu.async_copy(
                bf_seg.at[pl.ds(f * CAP2, CAP2)],
                fine_seg.at[pl.ds(fb * CAP2, CAP2)], sem))
        for d in descs:
            d.wait()


def _partition_fine(comp_src, comp_seg):
    return pl.kernel(
        _sc1b_body,
        out_type=[jax.ShapeDtypeStruct((NFBP * CAP2,), jnp.int32),
                  jax.ShapeDtypeStruct((NFBP * CAP2,), jnp.int32),
                  jax.ShapeDtypeStruct((NBK * FPC,), jnp.int32)],
        mesh=_MESH,
        scratch_types=[
            pltpu.VMEM((PSZ,), jnp.int32),
            pltpu.VMEM((PSZ,), jnp.int32),
            pltpu.VMEM((FPC * CAP2,), jnp.int32),
            pltpu.VMEM((FPC * CAP2,), jnp.int32),
            pltpu.VMEM((L,), jnp.int32),
            pltpu.SemaphoreType.DMA,
        ],
        compiler_params=pltpu.CompilerParams(use_tc_tiling_on_sc=False,
                                             needs_layout_passes=False),
    )(comp_src, comp_seg)


def _make_sc2_body(emit_counts):
    def body(*refs):
        if emit_counts:
            (fine_src, fine_seg, counts2, vals, outS, outC,
             idxA, idxB, segA, segB, rowsA, rowsB, ctb, acc, accC,
             semIA, semIB, semGA, semGB) = refs
        else:
            (fine_src, fine_seg, counts2, vals, outS,
             idxA, idxB, segA, segB, rowsA, rowsB, ctb, acc,
             semIA, semIB, semGA, semGB) = refs
        c = lax.axis_index("c")
        s = lax.axis_index("s")
        w = s * NC + c
        idx = (idxA, idxB)
        seg = (segA, segB)
        rows = (rowsA, rowsB)
        semI = (semIA, semIB)
        semG = (semGA, semGB)

        pltpu.sync_copy(counts2, ctb.at[pl.ds(0, NBK * FPC)])
        lanes = lax.iota(jnp.int32, L)
        lane = jnp.bitwise_and(w, L - 1)
        half = s >= (NS // 2)
        lanemask = lanes == lane
        e1_v = jnp.where(lanes == 0, jnp.float32(1.0), jnp.float32(0.0))

        def one_pass(ps, _):
            fb = ps * NW + w

            @pl.when(fb < NFB)
            def _():
                v1 = ctb[pl.ds(ps * NW, L)]
                v2 = ctb[pl.ds(ps * NW + L, L)]
                cv = jnp.where(half, v2, v1)
                cnt = jnp.max(jnp.where(lanemask, cv, 0))
                nch = (cnt + (K - 1)) // K
                base = fb * CAP2

                zv = jnp.zeros((L,), jnp.float32)

                def zloop(i, _):
                    acc[pl.ds(i * L, L)] = zv
                    return 0

                lax.fori_loop(0, ARW2 // L, zloop, 0)
                if emit_counts:
                    def zloop2(i, _):
                        accC[pl.ds(i * L, L)] = zv
                        return 0

                    lax.fori_loop(0, (AR2 + L) // L, zloop2, 0)

                def start_is(q, p):
                    o = base + q * K
                    pltpu.async_copy(fine_src.at[pl.ds(o, K)], idx[p],
                                     semI[p])
                    pltpu.async_copy(fine_seg.at[pl.ds(o, K)], seg[p],
                                     semI[p])

                def wait_is(p):
                    pltpu.make_async_copy(fine_src.at[pl.ds(0, K)], idx[p],
                                          semI[p]).wait()
                    pltpu.make_async_copy(fine_seg.at[pl.ds(0, K)], seg[p],
                                          semI[p]).wait()

                def start_g(p):
                    pltpu.async_copy(vals.at[idx[p]], rows[p], semG[p])

                def wait_g(p):
                    pltpu.make_async_copy(vals.at[idx[p]], rows[p],
                                          semG[p]).wait()

                def add_chunk(p):
                    def group(g):
                        sg = seg[p][pl.ds(g * L, L)]
                        so = sg * H
                        for ln in range(L):
                            o = so[ln]
                            e = g * L + ln
                            for j in range(H // L):
                                v = rows[p][e, pl.ds(j * L, L)]
                                plsc.addupdate(
                                    acc.at[pl.ds(o + j * L, L)], v)
                            if emit_counts:
                                plsc.addupdate(
                                    accC.at[pl.ds(sg[ln], L)], e1_v)

                    plsc.parallel_loop(0, K // L, 1, unroll=1)(group)

                @pl.when(nch > 0)
                def _():
                    start_is(0, 0)
                    wait_is(0)
                    start_g(0)

                @pl.when(nch > 1)
                def _():
                    start_is(1, 1)

                def piped(k2, _):
                    q = 2 * k2

                    @pl.when(q + 1 < nch)
                    def _():
                        wait_is(1)
                        start_g(1)

                    wait_g(0)
                    add_chunk(0)

                    @pl.when(q + 2 < nch)
                    def _():
                        start_is(q + 2, 0)
                        wait_is(0)
                        start_g(0)

                    @pl.when(q + 1 < nch)
                    def _():
                        wait_g(1)
                        add_chunk(1)

                    @pl.when(q + 3 < nch)
                    def _():
                        start_is(q + 3, 1)

                    return 0

                lax.fori_loop(0, (nch + 1) // 2, piped, 0)

                for r in range(R):
                    pltpu.sync_copy(
                        acc.at[pl.ds(r * FB * H, FB * H)],
                        outS.at[pl.ds((r * NPAD + fb * FB) * H, FB * H)])
                    if emit_counts:
                        pltpu.sync_copy(
                            accC.at[pl.ds(r * FB, FB)],
                            outC.at[pl.ds(r * NPAD + fb * FB, FB)])

            return 0

        lax.fori_loop(0, NPASS, one_pass, 0)

    return body


def _seg_sums_sc(vals, fine_src, fine_seg, counts2, emit_counts):
    cparams = pltpu.CompilerParams(use_tc_tiling_on_sc=False,
                                   needs_layout_passes=False)
    buf2 = lambda shape, dt: [pltpu.VMEM(shape, dt), pltpu.VMEM(shape, dt)]
    sems = [pltpu.SemaphoreType.DMA] * 4
    if emit_counts:
        S, cnt = pl.kernel(
            _make_sc2_body(True),
            out_type=[jax.ShapeDtypeStruct((R * NPAD * H,), jnp.float32),
                      jax.ShapeDtypeStruct((R * NPAD,), jnp.float32)],
            mesh=_MESH,
            scratch_types=(
                buf2((K,), jnp.int32) + buf2((K,), jnp.int32)
                + buf2((K, H), jnp.float32)
                + [pltpu.VMEM((NPASS * NW,), jnp.int32),
                   pltpu.VMEM((ARW2,), jnp.float32),
                   pltpu.VMEM((AR2 + L,), jnp.float32)]
                + sems),
            compiler_params=cparams,
        )(fine_src, fine_seg, counts2, vals)
        return (S.reshape(R, NPAD, H), cnt.reshape(R, NPAD, 1))
    S = pl.kernel(
        _make_sc2_body(False),
        out_type=jax.ShapeDtypeStruct((R * NPAD * H,), jnp.float32),
        mesh=_MESH,
        scratch_types=(
            buf2((K,), jnp.int32) + buf2((K,), jnp.int32)
            + buf2((K, H), jnp.float32)
            + [pltpu.VMEM((NPASS * NW,), jnp.int32),
               pltpu.VMEM((ARW2,), jnp.float32)]
            + sems),
        compiler_params=cparams,
    )(fine_src, fine_seg, counts2, vals)
    return S.reshape(R, NPAD, H), None


def _layer_a_body(S_ref, cnt_ref, x_ref, Wrel_ref, Wroot_ref, b_ref,
                  hpre_ref, stats_ref):
    i = pl.program_id(0)
    acc = jnp.dot(x_ref[...], Wroot_ref[...],
                  preferred_element_type=jnp.float32) + b_ref[...]
    for r in range(R):
        inv = 1.0 / jnp.maximum(cnt_ref[r], 1.0)
        acc += jnp.dot(S_ref[r] * inv, Wrel_ref[r],
                       preferred_element_type=jnp.float32)
    hpre_ref[...] = acc

    @pl.when(i == 0)
    def _():
        stats_ref[...] = jnp.zeros_like(stats_ref)

    cs = jnp.sum(acc, axis=0).reshape(1, H)
    cq = jnp.sum(acc * acc, axis=0).reshape(1, H)
    stats_ref[...] += jnp.concatenate(
        [cs, cq, jnp.zeros((6, H), jnp.float32)], axis=0)


def _layer_a(S, cnt, x, W_rel, W_root, b):
    return pl.pallas_call(
        _layer_a_body,
        grid=(NBLK,),
        in_specs=[
            pl.BlockSpec((R, NB, H), lambda i: (0, i, 0)),
            pl.BlockSpec((R, NB, 1), lambda i: (0, i, 0)),
            pl.BlockSpec((NB, H), lambda i: (i, 0)),
            pl.BlockSpec((R, F_IN, H), lambda i: (0, 0, 0)),
            pl.BlockSpec((F_IN, H), lambda i: (0, 0)),
            pl.BlockSpec((1, H), lambda i: (0, 0)),
        ],
        out_specs=[
            pl.BlockSpec((NB, H), lambda i: (i, 0)),
            pl.BlockSpec((8, H), lambda i: (0, 0)),
        ],
        out_shape=[
            jax.ShapeDtypeStruct((N, H), jnp.float32),
            jax.ShapeDtypeStruct((8, H), jnp.float32),
        ],
    )(S, cnt, x, W_rel, W_root, b.reshape(1, H))


def _layer_c_body(hpre_ref, stats_ref, gamma_ref, beta_ref, a_ref, h_ref):
    s = stats_ref[...]
    mu = s[0:1, :] * (1.0 / N)
    ex2 = s[1:2, :] * (1.0 / N)
    var = ex2 - mu * mu
    y = (hpre_ref[...] - mu) * lax.rsqrt(var + 1e-5) * gamma_ref[...] \
        + beta_ref[...]
    a = a_ref[0, 0]
    y = jnp.where(y >= 0, y, a * y)
    nrm = jnp.sqrt(jnp.sum(y * y, axis=1, keepdims=True))
    h_ref[...] = y / jnp.maximum(nrm, 1e-12)


def _layer_c(hpre, stats, gamma, beta, a):
    return pl.pallas_call(
        _layer_c_body,
        grid=(NBLK,),
        in_specs=[
            pl.BlockSpec((NB, H), lambda i: (i, 0)),
            pl.BlockSpec((8, H), lambda i: (0, 0)),
            pl.BlockSpec((1, H), lambda i: (0, 0)),
            pl.BlockSpec((1, H), lambda i: (0, 0)),
            pl.BlockSpec((1, 1), lambda i: (0, 0)),
        ],
        out_specs=pl.BlockSpec((NB, H), lambda i: (i, 0)),
        out_shape=jax.ShapeDtypeStruct((N, H), jnp.float32),
    )(hpre, stats, gamma.reshape(1, H), beta.reshape(1, H), a.reshape(1, 1))


def _pool_body(batch_ref, h_ref, fcW_ref, fcb_ref, outW_ref, outb_ref,
               o_ref, P_acc, c_acc):
    i = pl.program_id(0)

    @pl.when(i == 0)
    def _():
        P_acc[...] = jnp.zeros_like(P_acc)
        c_acc[...] = jnp.zeros_like(c_acc)

    iota_g = lax.broadcasted_iota(jnp.int32, (1, G), 1)
    onehot = (batch_ref[...] == iota_g).astype(jnp.float32)
    P_acc[...] += lax.dot_general(onehot, h_ref[...],
                                  (((0,), (0,)), ((), ())),
                                  preferred_element_type=jnp.float32)
    c_acc[...] += lax.dot_general(onehot, jnp.ones((NB, 1), jnp.float32),
                                  (((0,), (0,)), ((), ())),
                                  preferred_element_type=jnp.float32)

    @pl.when(i == NBLK - 1)
    def _():
        pooled = P_acc[...] / jnp.maximum(c_acc[...], 1.0)
        z = jnp.dot(pooled, fcW_ref[...],
                    preferred_element_type=jnp.float32) + fcb_ref[...]
        z = jnp.maximum(z, 0.0)
        o_ref[...] = jnp.dot(z, outW_ref[...],
                             preferred_element_type=jnp.float32) + outb_ref[...]


def _pool(batch2d, h, fc_W, fc_b, out_W, out_b):
    return pl.pallas_call(
        _pool_body,
        grid=(NBLK,),
        in_specs=[
            pl.BlockSpec((NB, 1), lambda i: (i, 0)),
            pl.BlockSpec((NB, H), lambda i: (i, 0)),
            pl.BlockSpec((H, H), lambda i: (0, 0)),
            pl.BlockSpec((1, H), lambda i: (0, 0)),
            pl.BlockSpec((H, 1), lambda i: (0, 0)),
            pl.BlockSpec((1, 1), lambda i: (0, 0)),
        ],
        out_specs=pl.BlockSpec((G, 1), lambda i: (0, 0)),
        out_shape=jax.ShapeDtypeStruct((G, 1), jnp.float32),
        scratch_shapes=[
            pltpu.VMEM((G, H), jnp.float32),
            pltpu.VMEM((G, 1), jnp.float32),
        ],
    )(batch2d, h, fc_W, fc_b.reshape(1, H), out_W, out_b.reshape(1, 1))


def _pad_chunks(a, fill):
    a2 = a.reshape(NW, EC)
    pad = jnp.full((NW, ECP - EC), fill, jnp.int32)
    return jnp.concatenate([a2, pad], axis=1).reshape(NW * ECP)


def kernel(x, edge_index, edge_type, batch, W_rel1, W_root1, b1, gamma1,
           beta1, a1, W_rel2, W_root2, b2, gamma2, beta2, a2, fc_W, fc_b,
           out_W, out_b):
    srcT = _pad_chunks(edge_index[0], 0)
    dstT = _pad_chunks(edge_index[1], 1 << 20)
    etT = _pad_chunks(edge_type, 0)

    comp_src, comp_seg, _counts = _compact_edges(srcT, dstT, etT)
    fine_src, fine_seg, counts2 = _partition_fine(comp_src, comp_seg)

    S1, cnt = _seg_sums_sc(x, fine_src, fine_seg, counts2, True)
    hpre1, stats1 = _layer_a(S1, cnt, x, W_rel1, W_root1, b1)
    h1 = _layer_c(hpre1, stats1, gamma1, beta1, a1)

    S2, _ = _seg_sums_sc(h1, fine_src, fine_seg, counts2, False)
    hpre2, stats2 = _layer_a(S2, cnt, h1, W_rel2, W_root2, b2)
    h2 = _layer_c(hpre2, stats2, gamma2, beta2, a2)

    return _pool(batch.reshape(N, 1), h2, fc_W, fc_b, out_W, out_b)
